# Initial kernel scaffold; baseline (speedup 1.0000x reference)
import jax
import jax.numpy as jnp
from jax.experimental import pallas as pl


def kernel(batch_inputs, entity_emb, relation_emb, rel_attention, conv_w, conv_b, fc_w, fc_b):
    raise NotImplementedError("write your pallas kernel here")


# speedup vs baseline: 3.6698x; 3.6698x over previous
import functools

import jax
import jax.numpy as jnp
from jax import lax
from jax.experimental import pallas as pl
from jax.experimental.pallas import tpu as pltpu
from jax.experimental.pallas import tpu_sc as plsc

_K = 8
_EMB_S = 64
_TOP_N = 4
_OUT_CH = 64
_NUM_REL = 500
_B = 16384
_D = _TOP_N * _EMB_S
_RPAD = 512


def _topk_body(att_ref, idx_ref, sum_ref):
    a = att_ref[...]
    m = jnp.max(a, axis=1, keepdims=True)
    e = jnp.exp(a - m)
    sm = e / jnp.sum(e, axis=1, keepdims=True)
    iota = lax.broadcasted_iota(jnp.int32, (_RPAD, _K), 1)
    masked = sm
    tot = jnp.zeros((_RPAD, 1), jnp.float32)
    cols = []
    for _ in range(_TOP_N):
        mx = jnp.max(masked, axis=1, keepdims=True)
        cand = jnp.where(masked == mx, iota, _K)
        am = jnp.min(cand, axis=1, keepdims=True)
        cols.append(am)
        tot = tot + mx
        masked = jnp.where(iota == am, -1.0, masked)
    idx_ref[...] = jnp.concatenate(cols, axis=1)
    sum_ref[...] = tot


def _rel_topk(att_pad):
    return pl.pallas_call(
        _topk_body,
        out_shape=(
            jax.ShapeDtypeStruct((_RPAD, _TOP_N), jnp.int32),
            jax.ShapeDtypeStruct((_RPAD, 1), jnp.float32),
        ),
    )(att_pad)


_NC = 2
_NS = 16
_NW = _NC * _NS
_PER_W = _B // _NW
_CH = 128
_NCHUNK = _PER_W // _CH
_L = 16
_G = _CH // _L


def _gather_body(hcol_hbm, rcol_hbm, tcol_hbm, tix_hbm, tsum_hbm, ent_hbm,
                 rel_hbm, headg_hbm, tailg_hbm, relg_hbm, atts_hbm,
                 tix_v, tsum_v, hcol_v, rcol_v, tcol_v, atts_v,
                 hidx_v, tidx_v, hrows_v, trows_v, relrows_v,
                 sem_h, sem_t, sem_r):
    wid = lax.axis_index("s") * _NC + lax.axis_index("c")
    pltpu.sync_copy(tix_hbm, tix_v)
    pltpu.sync_copy(tsum_hbm, tsum_v)
    lane = lax.iota(jnp.int32, 16)
    for c in range(_NCHUNK):
        base = wid * _PER_W + c * _CH
        pltpu.sync_copy(hcol_hbm.at[pl.ds(base, _CH)], hcol_v)
        pltpu.sync_copy(rcol_hbm.at[pl.ds(base, _CH)], rcol_v)
        pltpu.sync_copy(tcol_hbm.at[pl.ds(base, _CH)], tcol_v)
        for g in range(_G):
            h_vec = hcol_v[pl.ds(g * _L, _L)]
            r_vec = rcol_v[pl.ds(g * _L, _L)]
            t_vec = tcol_v[pl.ds(g * _L, _L)]
            av = plsc.load_gather(tsum_v, [r_vec])
            atts_v[pl.ds(g * _L, _L)] = av
            row = jnp.full((16,), g // 2, jnp.int32)
            for n in range(_TOP_N):
                tin = plsc.load_gather(tix_v, [r_vec * _TOP_N + n])
                col = lane * _TOP_N + ((g % 2) * 64 + n)
                plsc.store_scatter(hidx_v, [row, col], h_vec * _K + tin)
                plsc.store_scatter(tidx_v, [row, col], t_vec * _K + tin)
        copies = []
        for i in range(4):
            copies.append(pltpu.async_copy(
                ent_hbm.at[hidx_v.at[i]], hrows_v.at[pl.ds(i * _CH, _CH)],
                sem_h))
            copies.append(pltpu.async_copy(
                ent_hbm.at[tidx_v.at[i]], trows_v.at[pl.ds(i * _CH, _CH)],
                sem_t))
        copies.append(pltpu.async_copy(rel_hbm.at[rcol_v], relrows_v, sem_r))
        for cp in copies:
            cp.wait()
        pltpu.sync_copy(hrows_v, headg_hbm.at[pl.ds(base * _TOP_N, _CH * _TOP_N)])
        pltpu.sync_copy(trows_v, tailg_hbm.at[pl.ds(base * _TOP_N, _CH * _TOP_N)])
        pltpu.sync_copy(relrows_v, relg_hbm.at[pl.ds(base, _CH)])
        pltpu.sync_copy(atts_v, atts_hbm.at[pl.ds(base, _CH)])


def _sc_gather(hcol, rcol, tcol, tix_flat, tsum_flat, ent2, rel):
    mesh = plsc.VectorSubcoreMesh(core_axis_name="c", subcore_axis_name="s")
    run = functools.partial(
        pl.kernel,
        mesh=mesh,
        compiler_params=pltpu.CompilerParams(
            needs_layout_passes=False, use_tc_tiling_on_sc=False),
        out_type=(
            jax.ShapeDtypeStruct((_B * _TOP_N, _EMB_S), jnp.float32),
            jax.ShapeDtypeStruct((_B * _TOP_N, _EMB_S), jnp.float32),
            jax.ShapeDtypeStruct((_B, _D), jnp.float32),
            jax.ShapeDtypeStruct((_B,), jnp.float32),
        ),
        scratch_types=[
            pltpu.VMEM((_RPAD * _TOP_N,), jnp.int32),
            pltpu.VMEM((_RPAD,), jnp.float32),
            pltpu.VMEM((_CH,), jnp.int32),
            pltpu.VMEM((_CH,), jnp.int32),
            pltpu.VMEM((_CH,), jnp.int32),
            pltpu.VMEM((_CH,), jnp.float32),
            pltpu.VMEM((4, _CH), jnp.int32),
            pltpu.VMEM((4, _CH), jnp.int32),
            pltpu.VMEM((_CH * _TOP_N, _EMB_S), jnp.float32),
            pltpu.VMEM((_CH * _TOP_N, _EMB_S), jnp.float32),
            pltpu.VMEM((_CH, _D), jnp.float32),
            pltpu.SemaphoreType.DMA,
            pltpu.SemaphoreType.DMA,
            pltpu.SemaphoreType.DMA,
        ],
    )(_gather_body)
    return run(hcol, rcol, tcol, tix_flat, tsum_flat, ent2, rel)


_BB = 512
_NBLK = _B // _BB


def _dense_body(w_ref, cb_ref, fcb_ref, h_ref, r_ref, t_ref, fc_ref,
                atts_ref, out_ref, loss_ref):
    i = pl.program_id(0)
    h = h_ref[...]
    r = r_ref[...]
    t = t_ref[...]
    acc = jnp.zeros((_BB, _D), jnp.float32)
    for o in range(_OUT_CH):
        z = h * w_ref[o, 0] + r * w_ref[o, 1] + t * w_ref[o, 2] + cb_ref[o]
        acc = acc + jnp.maximum(z, 0.0) * fc_ref[o:o + 1, :]
    out_ref[...] = jnp.sum(acc, axis=1, keepdims=True) + fcb_ref[0]
    prev = jnp.where(i == 0, jnp.zeros((1, 1), jnp.float32), loss_ref[...])
    tot = prev + jnp.sum(atts_ref[...])
    loss_ref[...] = jnp.where(i == _NBLK - 1, 1.0 - tot / _B, tot)


def _dense(w2, cb, fcb, headg, relg, tailg, fc2, atts3):
    return pl.pallas_call(
        _dense_body,
        grid=(_NBLK,),
        in_specs=[
            pl.BlockSpec(memory_space=pltpu.SMEM),
            pl.BlockSpec(memory_space=pltpu.SMEM),
            pl.BlockSpec(memory_space=pltpu.SMEM),
            pl.BlockSpec((_BB, _D), lambda i: (i, 0)),
            pl.BlockSpec((_BB, _D), lambda i: (i, 0)),
            pl.BlockSpec((_BB, _D), lambda i: (i, 0)),
            pl.BlockSpec((_OUT_CH, _D), lambda i: (0, 0)),
            pl.BlockSpec((1, 1, _BB), lambda i: (i, 0, 0)),
        ],
        out_specs=(
            pl.BlockSpec((_BB, 1), lambda i: (i, 0)),
            pl.BlockSpec((1, 1), lambda i: (0, 0)),
        ),
        out_shape=(
            jax.ShapeDtypeStruct((_B, 1), jnp.float32),
            jax.ShapeDtypeStruct((1, 1), jnp.float32),
        ),
    )(w2, cb, fcb, headg, relg, tailg, fc2, atts3)


def kernel(batch_inputs, entity_emb, relation_emb, rel_attention, conv_w,
           conv_b, fc_w, fc_b):
    hcol = batch_inputs[:, 0].astype(jnp.int32)
    rcol = batch_inputs[:, 1].astype(jnp.int32)
    tcol = batch_inputs[:, 2].astype(jnp.int32)
    att_pad = jnp.zeros((_RPAD, _K), jnp.float32).at[:_NUM_REL].set(rel_attention)
    topidx, topsum = _rel_topk(att_pad)
    ent2 = entity_emb.reshape(-1, _EMB_S)
    headg, tailg, relg, atts = _sc_gather(
        hcol, rcol, tcol, topidx.reshape(-1), topsum.reshape(-1),
        ent2, relation_emb)
    w2 = conv_w.reshape(_OUT_CH, 3)
    fc2 = fc_w.reshape(_OUT_CH, _D)
    out, loss = _dense(
        w2, conv_b, fc_b,
        headg.reshape(_B, _D), relg, tailg.reshape(_B, _D), fc2,
        atts.reshape(_NBLK, 1, _BB))
    return (out, loss.reshape(()))


# speedup vs baseline: 4.9825x; 1.3577x over previous
import functools

import jax
import jax.numpy as jnp
from jax import lax
from jax.experimental import pallas as pl
from jax.experimental.pallas import tpu as pltpu
from jax.experimental.pallas import tpu_sc as plsc

_K = 8
_EMB_S = 64
_TOP_N = 4
_OUT_CH = 64
_NUM_REL = 500
_B = 16384
_D = _TOP_N * _EMB_S
_DE = _K * _EMB_S
_RPAD = 512
_OHW = 128


def _topk_body(att_ref, fc_ref, cb_ref, oh_ref, cc_ref):
    a = att_ref[...]
    m = jnp.max(a, axis=1, keepdims=True)
    e = jnp.exp(a - m)
    sm = e / jnp.sum(e, axis=1, keepdims=True)
    iota = lax.broadcasted_iota(jnp.int32, (_RPAD, _K), 1)
    masked = sm
    tot = jnp.zeros((_RPAD, 1), jnp.float32)
    cols = []
    for _ in range(_TOP_N):
        mx = jnp.max(masked, axis=1, keepdims=True)
        cand = jnp.where(masked == mx, iota, _K)
        am = jnp.min(cand, axis=1, keepdims=True)
        cols.append(am)
        tot = tot + mx
        masked = jnp.where(iota == am, -1.0, masked)
    wide = lax.broadcasted_iota(jnp.int32, (_RPAD, _OHW), 1)
    oh = jnp.zeros((_RPAD, _OHW), jnp.float32)
    for n in range(_TOP_N):
        oh = oh + jnp.where(wide == (n * _K + cols[n]), 1.0, 0.0)
    oh = oh + jnp.where(wide == _TOP_N * _K, tot, 0.0)
    oh_ref[...] = oh
    cc_ref[...] = jnp.sum(fc_ref[...] * cb_ref[...],
                          axis=(0, 1))[None, None]


def _rel_topk(att_pad, fc2, cb):
    return pl.pallas_call(
        _topk_body,
        out_shape=(
            jax.ShapeDtypeStruct((_RPAD, _OHW), jnp.float32),
            jax.ShapeDtypeStruct((1, 1), jnp.float32),
        ),
    )(att_pad, fc2, cb.reshape(_OUT_CH, 1))


_NC = 2
_NS = 16
_NW = _NC * _NS
_PER_W = _B // _NW
_CH = 32
_NCHUNK = _PER_W // _CH


def _gather_body(hcol_hbm, rcol_hbm, tcol_hbm, oh_hbm, ent_hbm, rel_hbm,
                 heado_hbm, tailo_hbm, relg_hbm, ohg_hbm,
                 hcol_v, rcol_v, tcol_v, hrows_v, trows_v, relrows_v, ohrows_v,
                 sem_h, sem_t, sem_r, sem_g, sem_o0, sem_o1):
    wid = lax.axis_index("s") * _NC + lax.axis_index("c")
    sem_o = (sem_o0, sem_o1)
    wbase = wid * _PER_W
    pltpu.sync_copy(hcol_hbm.at[pl.ds(wbase, _PER_W)], hcol_v)
    pltpu.sync_copy(rcol_hbm.at[pl.ds(wbase, _PER_W)], rcol_v)
    pltpu.sync_copy(tcol_hbm.at[pl.ds(wbase, _PER_W)], tcol_v)

    def fire_gathers(c, b):
        sl = pl.ds(c * _CH, _CH)
        return [
            pltpu.async_copy(ent_hbm.at[hcol_v.at[sl]], hrows_v.at[b], sem_h),
            pltpu.async_copy(ent_hbm.at[tcol_v.at[sl]], trows_v.at[b], sem_t),
            pltpu.async_copy(rel_hbm.at[rcol_v.at[sl]], relrows_v.at[b], sem_r),
            pltpu.async_copy(oh_hbm.at[rcol_v.at[sl]], ohrows_v.at[b], sem_g),
        ]

    def fire_out(c, b):
        base = wbase + c * _CH
        s = sem_o[b]
        sl = pl.ds(base, _CH)
        return [
            pltpu.async_copy(hrows_v.at[b], heado_hbm.at[sl], s),
            pltpu.async_copy(trows_v.at[b], tailo_hbm.at[sl], s),
            pltpu.async_copy(relrows_v.at[b], relg_hbm.at[sl], s),
            pltpu.async_copy(ohrows_v.at[b], ohg_hbm.at[sl], s),
        ]

    gath = fire_gathers(0, 0)
    out_pending = [None, None]
    for c in range(_NCHUNK):
        b = c % 2
        for h in gath:
            h.wait()
        oc = fire_out(c, b)
        if c + 1 < _NCHUNK:
            if out_pending[1 - b] is not None:
                for h in out_pending[1 - b]:
                    h.wait()
                out_pending[1 - b] = None
            gath = fire_gathers(c + 1, 1 - b)
        out_pending[b] = oc
    for p in out_pending:
        if p is not None:
            for h in p:
                h.wait()


def _sc_gather(hcol, rcol, tcol, ohtab, ent, rel):
    mesh = plsc.VectorSubcoreMesh(core_axis_name="c", subcore_axis_name="s")
    run = functools.partial(
        pl.kernel,
        mesh=mesh,
        out_type=(
            jax.ShapeDtypeStruct((_B, _DE), jnp.float32),
            jax.ShapeDtypeStruct((_B, _DE), jnp.float32),
            jax.ShapeDtypeStruct((_B, _D), jnp.float32),
            jax.ShapeDtypeStruct((_B, _OHW), jnp.float32),
        ),
        scratch_types=[
            pltpu.VMEM((_PER_W,), jnp.int32),
            pltpu.VMEM((_PER_W,), jnp.int32),
            pltpu.VMEM((_PER_W,), jnp.int32),
            pltpu.VMEM((2, _CH, _DE), jnp.float32),
            pltpu.VMEM((2, _CH, _DE), jnp.float32),
            pltpu.VMEM((2, _CH, _D), jnp.float32),
            pltpu.VMEM((2, _CH, _OHW), jnp.float32),
            pltpu.SemaphoreType.DMA,
            pltpu.SemaphoreType.DMA,
            pltpu.SemaphoreType.DMA,
            pltpu.SemaphoreType.DMA,
            pltpu.SemaphoreType.DMA,
            pltpu.SemaphoreType.DMA,
        ],
    )(_gather_body)
    return run(hcol, rcol, tcol, ohtab, ent, rel)


_BB = 512
_NBLK = _B // _BB


def _dense_body(w_ref, cb_ref, fcb_ref, cc_ref, h_ref, r_ref, t_ref, oh_ref,
                fc_ref, out_ref, loss_ref):
    i = pl.program_id(0)
    hfull = h_ref[...]
    tfull = t_ref[...]
    rg = r_ref[...]
    oh = oh_ref[...]
    hsegs, tsegs = [], []
    for n in range(_TOP_N):
        ha = jnp.zeros((_BB, _EMB_S), jnp.float32)
        ta = jnp.zeros((_BB, _EMB_S), jnp.float32)
        for k in range(_K):
            m = oh[:, n * _K + k:n * _K + k + 1]
            ha = ha + hfull[:, k * _EMB_S:(k + 1) * _EMB_S] * m
            ta = ta + tfull[:, k * _EMB_S:(k + 1) * _EMB_S] * m
        hsegs.append(ha)
        tsegs.append(ta)
    hg = jnp.concatenate(hsegs, axis=1)
    tg = jnp.concatenate(tsegs, axis=1)
    acc = jnp.zeros((_BB, _D), jnp.float32)
    for o in range(_OUT_CH):
        z = hg * w_ref[o, 0] + rg * w_ref[o, 1] + tg * w_ref[o, 2]
        acc = acc + jnp.maximum(z, -cb_ref[o]) * fc_ref[o:o + 1, :]
    out_ref[...] = (jnp.sum(acc, axis=1, keepdims=True)
                    + (cc_ref[0, 0] + fcb_ref[0]))
    prev = jnp.where(i == 0, jnp.zeros((1, 1), jnp.float32), loss_ref[...])
    tot = prev + jnp.sum(oh[:, _TOP_N * _K:_TOP_N * _K + 1])
    loss_ref[...] = jnp.where(i == _NBLK - 1, 1.0 - tot / _B, tot)


def _dense(w2, cb, fcb, cconst, heado, relg, tailo, ohg, fc2):
    return pl.pallas_call(
        _dense_body,
        grid=(_NBLK,),
        in_specs=[
            pl.BlockSpec(memory_space=pltpu.SMEM),
            pl.BlockSpec(memory_space=pltpu.SMEM),
            pl.BlockSpec(memory_space=pltpu.SMEM),
            pl.BlockSpec(memory_space=pltpu.SMEM),
            pl.BlockSpec((_BB, _DE), lambda i: (i, 0)),
            pl.BlockSpec((_BB, _D), lambda i: (i, 0)),
            pl.BlockSpec((_BB, _DE), lambda i: (i, 0)),
            pl.BlockSpec((_BB, _OHW), lambda i: (i, 0)),
            pl.BlockSpec((_OUT_CH, _D), lambda i: (0, 0)),
        ],
        out_specs=(
            pl.BlockSpec((_BB, 1), lambda i: (i, 0)),
            pl.BlockSpec((1, 1), lambda i: (0, 0)),
        ),
        out_shape=(
            jax.ShapeDtypeStruct((_B, 1), jnp.float32),
            jax.ShapeDtypeStruct((1, 1), jnp.float32),
        ),
    )(w2, cb, fcb, cconst, heado, relg, tailo, ohg, fc2)


def kernel(batch_inputs, entity_emb, relation_emb, rel_attention, conv_w,
           conv_b, fc_w, fc_b):
    hcol = batch_inputs[:, 0].astype(jnp.int32)
    rcol = batch_inputs[:, 1].astype(jnp.int32)
    tcol = batch_inputs[:, 2].astype(jnp.int32)
    att_pad = jnp.zeros((_RPAD, _K), jnp.float32).at[:_NUM_REL].set(rel_attention)
    w2 = conv_w.reshape(_OUT_CH, 3)
    fc2 = fc_w.reshape(_OUT_CH, _D)
    ohtab, cconst = _rel_topk(att_pad, fc2, conv_b)
    heado, tailo, relg, ohg = _sc_gather(
        hcol, rcol, tcol, ohtab, entity_emb, relation_emb)
    out, loss = _dense(w2, conv_b, fc_b, cconst, heado, relg, tailo, ohg, fc2)
    return (out, loss.reshape(()))


# speedup vs baseline: 5.8815x; 1.1804x over previous
import functools

import jax
import jax.numpy as jnp
from jax import lax
from jax.experimental import pallas as pl
from jax.experimental.pallas import tpu as pltpu
from jax.experimental.pallas import tpu_sc as plsc

_K = 8
_EMB_S = 64
_TOP_N = 4
_OUT_CH = 64
_NUM_REL = 500
_B = 16384
_D = _TOP_N * _EMB_S
_DE = _K * _EMB_S
_RPAD = 512
_TW = 128


def _topk_body(att_ref, fc_ref, cb_ref, tix_ref, cc_ref):
    a = att_ref[...]
    m = jnp.max(a, axis=1, keepdims=True)
    e = jnp.exp(a - m)
    sm = e / jnp.sum(e, axis=1, keepdims=True)
    iota = lax.broadcasted_iota(jnp.int32, (_RPAD, _K), 1)
    masked = sm
    tot = jnp.zeros((_RPAD, 1), jnp.float32)
    cols = []
    for _ in range(_TOP_N):
        mx = jnp.max(masked, axis=1, keepdims=True)
        cand = jnp.where(masked == mx, iota, _K)
        am = jnp.min(cand, axis=1, keepdims=True)
        cols.append(am)
        tot = tot + mx
        masked = jnp.where(iota == am, -1.0, masked)
    cols.append(lax.bitcast_convert_type(tot, jnp.int32))
    cols.append(jnp.zeros((_RPAD, _TW - _TOP_N - 1), jnp.int32))
    tix_ref[...] = jnp.concatenate(cols, axis=1)
    cc_ref[...] = jnp.sum(fc_ref[...] * cb_ref[...],
                          axis=(0, 1))[None, None]


def _rel_topk(att_pad, fc2, cb):
    return pl.pallas_call(
        _topk_body,
        out_shape=(
            jax.ShapeDtypeStruct((_RPAD, _TW), jnp.int32),
            jax.ShapeDtypeStruct((1, 1), jnp.float32),
        ),
    )(att_pad, fc2, cb.reshape(_OUT_CH, 1))


_NC = 2
_NS = 16
_NW = _NC * _NS
_PER_W = _B // _NW
_CH = 32
_NCHUNK = _PER_W // _CH


def _gather_body(hcol_hbm, rcol_hbm, tcol_hbm, tix_hbm, ent_hbm, rel_hbm,
                 headg_hbm, tailg_hbm, relg_hbm, tixg_hbm,
                 hcol_v, rcol_v, tcol_v, hrows_v, trows_v, relrows_v,
                 tixrows_v, hout_v, tout_v,
                 sem_h, sem_t, sem_r, sem_x, sem_o0, sem_o1):
    wid = lax.axis_index("s") * _NC + lax.axis_index("c")
    sem_o = (sem_o0, sem_o1)
    wbase = wid * _PER_W
    pltpu.sync_copy(hcol_hbm.at[pl.ds(wbase, _PER_W)], hcol_v)
    pltpu.sync_copy(rcol_hbm.at[pl.ds(wbase, _PER_W)], rcol_v)
    pltpu.sync_copy(tcol_hbm.at[pl.ds(wbase, _PER_W)], tcol_v)

    def fire_gathers(c, b):
        sl = pl.ds(c * _CH, _CH)
        return [
            pltpu.async_copy(ent_hbm.at[hcol_v.at[sl]], hrows_v.at[b], sem_h),
            pltpu.async_copy(ent_hbm.at[tcol_v.at[sl]], trows_v.at[b], sem_t),
            pltpu.async_copy(rel_hbm.at[rcol_v.at[sl]], relrows_v.at[b], sem_r),
            pltpu.async_copy(tix_hbm.at[rcol_v.at[sl]], tixrows_v.at[b], sem_x),
        ]

    def fire_out(c, b):
        base = wbase + c * _CH
        s = sem_o[b]
        sl = pl.ds(base, _CH)
        return [
            pltpu.async_copy(hout_v.at[b], headg_hbm.at[sl], s),
            pltpu.async_copy(tout_v.at[b], tailg_hbm.at[sl], s),
            pltpu.async_copy(relrows_v.at[b], relg_hbm.at[sl], s),
            pltpu.async_copy(tixrows_v.at[b], tixg_hbm.at[sl], s),
        ]

    def select(b):
        def body(e, carry):
            v = tixrows_v.at[b][e, pl.ds(0, 16)]
            for n in range(_TOP_N):
                s = v[n]
                sb = s * _EMB_S
                nb = n * _EMB_S
                for j in range(_EMB_S // 16):
                    hout_v.at[b][e, pl.ds(nb + j * 16, 16)] = (
                        hrows_v.at[b][e, pl.ds(sb + j * 16, 16)])
                    tout_v.at[b][e, pl.ds(nb + j * 16, 16)] = (
                        trows_v.at[b][e, pl.ds(sb + j * 16, 16)])
            return carry

        lax.fori_loop(0, _CH, body, 0)

    gath = fire_gathers(0, 0)
    out_pending = [None, None]
    for c in range(_NCHUNK):
        b = c % 2
        for h in gath:
            h.wait()
        if c + 1 < _NCHUNK:
            if out_pending[1 - b] is not None:
                for h in out_pending[1 - b]:
                    h.wait()
                out_pending[1 - b] = None
            gath = fire_gathers(c + 1, 1 - b)
        select(b)
        out_pending[b] = fire_out(c, b)
    for p in out_pending:
        if p is not None:
            for h in p:
                h.wait()


def _sc_gather(hcol, rcol, tcol, tix16, ent, rel):
    mesh = plsc.VectorSubcoreMesh(core_axis_name="c", subcore_axis_name="s")
    run = functools.partial(
        pl.kernel,
        mesh=mesh,
        out_type=(
            jax.ShapeDtypeStruct((_B, _D), jnp.float32),
            jax.ShapeDtypeStruct((_B, _D), jnp.float32),
            jax.ShapeDtypeStruct((_B, _D), jnp.float32),
            jax.ShapeDtypeStruct((_B, _TW), jnp.int32),
        ),
        scratch_types=[
            pltpu.VMEM((_PER_W,), jnp.int32),
            pltpu.VMEM((_PER_W,), jnp.int32),
            pltpu.VMEM((_PER_W,), jnp.int32),
            pltpu.VMEM((2, _CH, _DE), jnp.float32),
            pltpu.VMEM((2, _CH, _DE), jnp.float32),
            pltpu.VMEM((2, _CH, _D), jnp.float32),
            pltpu.VMEM((2, _CH, _TW), jnp.int32),
            pltpu.VMEM((2, _CH, _D), jnp.float32),
            pltpu.VMEM((2, _CH, _D), jnp.float32),
            pltpu.SemaphoreType.DMA,
            pltpu.SemaphoreType.DMA,
            pltpu.SemaphoreType.DMA,
            pltpu.SemaphoreType.DMA,
            pltpu.SemaphoreType.DMA,
            pltpu.SemaphoreType.DMA,
        ],
    )(_gather_body)
    return run(hcol, rcol, tcol, tix16, ent, rel)


_BB = 512
_NBLK = _B // _BB


def _dense_body(w_ref, cb_ref, fcb_ref, cc_ref, h_ref, r_ref, t_ref,
                tix_ref, fc_ref, out_ref, loss_ref):
    i = pl.program_id(0)
    h = h_ref[...]
    r = r_ref[...]
    t = t_ref[...]
    acc = jnp.zeros((_BB, _D), jnp.float32)
    for o in range(_OUT_CH):
        z = h * w_ref[o, 0] + r * w_ref[o, 1] + t * w_ref[o, 2]
        acc = acc + jnp.maximum(z, -cb_ref[o]) * fc_ref[o:o + 1, :]
    out_ref[...] = (jnp.sum(acc, axis=1, keepdims=True)
                    + (cc_ref[0, 0] + fcb_ref[0]))
    atts = lax.bitcast_convert_type(
        tix_ref[:, _TOP_N:_TOP_N + 1], jnp.float32)
    prev = jnp.where(i == 0, jnp.zeros((1, 1), jnp.float32), loss_ref[...])
    tot = prev + jnp.sum(atts)
    loss_ref[...] = jnp.where(i == _NBLK - 1, 1.0 - tot / _B, tot)


def _dense(w2, cb, fcb, cconst, headg, relg, tailg, tixg, fc2):
    return pl.pallas_call(
        _dense_body,
        grid=(_NBLK,),
        in_specs=[
            pl.BlockSpec(memory_space=pltpu.SMEM),
            pl.BlockSpec(memory_space=pltpu.SMEM),
            pl.BlockSpec(memory_space=pltpu.SMEM),
            pl.BlockSpec(memory_space=pltpu.SMEM),
            pl.BlockSpec((_BB, _D), lambda i: (i, 0)),
            pl.BlockSpec((_BB, _D), lambda i: (i, 0)),
            pl.BlockSpec((_BB, _D), lambda i: (i, 0)),
            pl.BlockSpec((_BB, _TW), lambda i: (i, 0)),
            pl.BlockSpec((_OUT_CH, _D), lambda i: (0, 0)),
        ],
        out_specs=(
            pl.BlockSpec((_BB, 1), lambda i: (i, 0)),
            pl.BlockSpec((1, 1), lambda i: (0, 0)),
        ),
        out_shape=(
            jax.ShapeDtypeStruct((_B, 1), jnp.float32),
            jax.ShapeDtypeStruct((1, 1), jnp.float32),
        ),
    )(w2, cb, fcb, cconst, headg, relg, tailg, tixg, fc2)


def kernel(batch_inputs, entity_emb, relation_emb, rel_attention, conv_w,
           conv_b, fc_w, fc_b):
    hcol = batch_inputs[:, 0].astype(jnp.int32)
    rcol = batch_inputs[:, 1].astype(jnp.int32)
    tcol = batch_inputs[:, 2].astype(jnp.int32)
    att_pad = jnp.zeros((_RPAD, _K), jnp.float32).at[:_NUM_REL].set(rel_attention)
    w2 = conv_w.reshape(_OUT_CH, 3)
    fc2 = fc_w.reshape(_OUT_CH, _D)
    tix16, cconst = _rel_topk(att_pad, fc2, conv_b)
    headg, tailg, relg, tixg = _sc_gather(
        hcol, rcol, tcol, tix16, entity_emb, relation_emb)
    out, loss = _dense(w2, conv_b, fc_b, cconst, headg, relg, tailg, tixg, fc2)
    return (out, loss.reshape(()))
